# baseline (speedup 1.0000x reference)
import functools

import numpy as np
import jax
import jax.numpy as jnp
from jax import lax
from jax.experimental import pallas as pl
from jax.experimental.pallas import tpu as pltpu
from jax.experimental.pallas import tpu_sc as plsc

H = 256
HL = 64
M = 8
L = 4
NG = 4096
NS = 4096
NV = 8192
NQ = NS + NV
R0 = 0.05
R1 = 0.25
R1SQ = R1 * R1
KMAX = 32
BIG = 1e30

NTILES = 32
QPT = NQ // NTILES
CHUNKS = NG // 16
STAGE = NG + 32


def _sc_ball_body(gx_h, gy_h, gz_h, qx_h, qy_h, qz_h,
                  onx_h, ony_h, onz_h, od2_h,
                  gx, gy, gz, qx, qy, qz,
                  obx, oby, obz, obd, sd, si):
    cid = lax.axis_index("c")
    sid = lax.axis_index("s")
    wid = sid * 2 + cid
    base = wid * QPT

    pltpu.sync_copy(gx_h, gx)
    pltpu.sync_copy(gy_h, gy)
    pltpu.sync_copy(gz_h, gz)
    pltpu.sync_copy(qx_h.at[pl.ds(base, QPT)], qx.at[pl.ds(0, QPT)])
    pltpu.sync_copy(qy_h.at[pl.ds(base, QPT)], qy.at[pl.ds(0, QPT)])
    pltpu.sync_copy(qz_h.at[pl.ds(base, QPT)], qz.at[pl.ds(0, QPT)])

    iota16 = lax.iota(jnp.int32, 16)

    def per_query(qi, carry):
        qsl = pl.ds(qi, 16)
        vqx = jnp.full((16,), qx[qsl][0])
        vqy = jnp.full((16,), qy[qsl][0])
        vqz = jnp.full((16,), qz[qsl][0])

        def scan_chunk(cc, off):
            sl = pl.ds(cc * 16, 16)
            dx = gx[sl] - vqx
            dy = gy[sl] - vqy
            dz = gz[sl] - vqz
            d2 = dx * dx + dy * dy + dz * dz
            m = d2 <= R1SQ
            cum = plsc.cumsum(m.astype(jnp.int32))
            pos = cum + (off - 1)
            plsc.store_scatter(sd, [pos], d2, mask=m)
            plsc.store_scatter(si, [pos], iota16 + cc * 16, mask=m)
            return off + cum[15]

        n = lax.fori_loop(0, CHUNKS, scan_chunk, 0, unroll=4)

        sd[pl.ds(n, 16)] = jnp.full((16,), BIG)
        si[pl.ds(n, 16)] = jnp.zeros((16,), jnp.int32)
        nchunks = (n + 15) // 16

        def merge_chunk(cc, buf):
            a0d, a0i, a1d, a1i = buf
            cd = sd[pl.ds(cc * 16, 16)]
            ci = si[pl.ds(cc * 16, 16)]
            t32 = jnp.full((16,), jnp.max(a1d))
            nlt = plsc.all_reduce_population_count(cd < t32)

            def do_merge(args):
                a0d, a0i, a1d, a1i, cd, ci = args
                cd, ci = plsc.sort_key_val(cd, ci)
                rcd = lax.rev(cd, (0,))
                rci = lax.rev(ci, (0,))
                take = a1d <= rcd
                kd = jnp.where(take, a1d, rcd)
                ki = jnp.where(take, a1i, rci)
                kd, ki = plsc.sort_key_val(kd, ki)
                rkd = lax.rev(kd, (0,))
                rki = lax.rev(ki, (0,))
                t = a0d <= rkd
                ld = jnp.where(t, a0d, rkd)
                li = jnp.where(t, a0i, rki)
                hd = jnp.where(t, rkd, a0d)
                hi = jnp.where(t, rki, a0i)
                a0d, a0i = plsc.sort_key_val(ld, li)
                a1d, a1i = plsc.sort_key_val(hd, hi)
                return (a0d, a0i, a1d, a1i)

            return lax.cond(nlt[0] > 0, do_merge,
                            lambda args: (args[0], args[1], args[2], args[3]),
                            (a0d, a0i, a1d, a1i, cd, ci))

        init = (jnp.full((16,), BIG), jnp.zeros((16,), jnp.int32),
                jnp.full((16,), BIG), jnp.zeros((16,), jnp.int32))
        a0d, a0i, a1d, a1i = lax.fori_loop(0, nchunks, merge_chunk, init)

        ob = pl.ds(qi * 32, 16)
        ob2 = pl.ds(qi * 32 + 16, 16)
        obx[ob] = plsc.load_gather(gx, [a0i])
        obx[ob2] = plsc.load_gather(gx, [a1i])
        oby[ob] = plsc.load_gather(gy, [a0i])
        oby[ob2] = plsc.load_gather(gy, [a1i])
        obz[ob] = plsc.load_gather(gz, [a0i])
        obz[ob2] = plsc.load_gather(gz, [a1i])
        obd[ob] = a0d
        obd[ob2] = a1d
        return carry

    lax.fori_loop(0, QPT, per_query, 0)

    out_sl = pl.ds(base * 32, QPT * 32)
    pltpu.sync_copy(obx, onx_h.at[out_sl])
    pltpu.sync_copy(oby, ony_h.at[out_sl])
    pltpu.sync_copy(obz, onz_h.at[out_sl])
    pltpu.sync_copy(obd, od2_h.at[out_sl])


_sc_ball = pl.kernel(
    _sc_ball_body,
    out_type=tuple(jax.ShapeDtypeStruct((NQ * 32,), jnp.float32)
                   for _ in range(4)),
    mesh=plsc.VectorSubcoreMesh(core_axis_name="c", subcore_axis_name="s"),
    compiler_params=pltpu.CompilerParams(needs_layout_passes=False),
    scratch_types=[
        pltpu.VMEM((NG,), jnp.float32),
        pltpu.VMEM((NG,), jnp.float32),
        pltpu.VMEM((NG,), jnp.float32),
        pltpu.VMEM((QPT + 16,), jnp.float32),
        pltpu.VMEM((QPT + 16,), jnp.float32),
        pltpu.VMEM((QPT + 16,), jnp.float32),
        pltpu.VMEM((QPT * 32,), jnp.float32),
        pltpu.VMEM((QPT * 32,), jnp.float32),
        pltpu.VMEM((QPT * 32,), jnp.float32),
        pltpu.VMEM((QPT * 32,), jnp.float32),
        pltpu.VMEM((STAGE,), jnp.float32),
        pltpu.VMEM((STAGE,), jnp.int32),
    ],
)


FFP = 64


def _fourier(x, nrows):
    ii = lax.broadcasted_iota(jnp.int32, (1, M), 1)
    freqs = lax.shift_left(1, ii).astype(jnp.float32) * np.float32(np.pi)
    parts = [x]
    for d in range(3):
        ang = x[:, d:d + 1] * freqs
        parts.append(jnp.sin(ang))
        parts.append(jnp.cos(ang))
    parts.append(jnp.zeros((nrows, FFP - 51), jnp.float32))
    return jnp.concatenate(parts, axis=1)


GB = 512


def _ctx_body(geo_ref, bc_ref, wgeo_ref, wbc_ref, out_ref, acc_ref):
    i = pl.program_id(0)

    @pl.when(i == 0)
    def _():
        bc = bc_ref[...]
        hb = jnp.maximum(
            jnp.dot(bc, wbc_ref[...], preferred_element_type=jnp.float32), 0.0)
        acc_ref[...] = jnp.sum(hb, axis=0, keepdims=True) * 0.25

    ff = _fourier(geo_ref[...], GB)
    emb = jnp.dot(ff, wgeo_ref[...], preferred_element_type=jnp.float32)
    acc_ref[...] += jnp.sum(emb, axis=0, keepdims=True) * (1.0 / NG)

    @pl.when(i == pl.num_programs(0) - 1)
    def _():
        out_ref[...] = acc_ref[...]


_ctx = pl.pallas_call(
    _ctx_body,
    grid=(NG // GB,),
    in_specs=[
        pl.BlockSpec((GB, 3), lambda i: (i, 0)),
        pl.BlockSpec((8, 2), lambda i: (0, 0)),
        pl.BlockSpec((FFP, H), lambda i: (0, 0)),
        pl.BlockSpec((2, H), lambda i: (0, 0)),
    ],
    out_specs=pl.BlockSpec((1, H), lambda i: (0, 0)),
    out_shape=jax.ShapeDtypeStruct((1, H), jnp.float32),
    scratch_shapes=[pltpu.VMEM((1, H), jnp.float32)],
    compiler_params=pltpu.CompilerParams(
        dimension_semantics=("arbitrary",)),
)


QB = 256


def _trunk_body(pts_ref, nx_ref, ny_ref, nz_ref, nd2_ref, ctx_ref,
                w01_ref, wproj_ref, wpt_ref, wb1_ref, wb2_ref,
                whv_ref, bhv_ref, whs_ref, bhs_ref,
                ov_ref, os_ref):
    pts = pts_ref[...]
    relx = nx_ref[...] - pts[:, 0:1]
    rely = ny_ref[...] - pts[:, 1:2]
    relz = nz_ref[...] - pts[:, 2:3]
    dist = jnp.sqrt(nd2_ref[...])
    m0 = dist <= R0
    m1 = dist <= R1

    w01 = w01_ref[...]
    acc0 = jnp.zeros((QB, HL), jnp.float32)
    acc1 = jnp.zeros((QB, HL), jnp.float32)
    for s in range(KMAX):
        loc = jnp.concatenate(
            [relx[:, s:s + 1], rely[:, s:s + 1], relz[:, s:s + 1],
             dist[:, s:s + 1]], axis=1)
        enc = jnp.maximum(
            jnp.dot(loc, w01, preferred_element_type=jnp.float32), 0.0)
        mm1 = jnp.broadcast_to(m1[:, s:s + 1], (QB, HL))
        acc1 = jnp.maximum(acc1, jnp.where(mm1, enc[:, HL:], 0.0))
        if s < 8:
            mm0 = jnp.broadcast_to(m0[:, s:s + 1], (QB, HL))
            acc0 = jnp.maximum(acc0, jnp.where(mm0, enc[:, :HL], 0.0))

    pooled = jnp.dot(jnp.concatenate([acc0, acc1], axis=1), wproj_ref[...],
                     preferred_element_type=jnp.float32)
    ff = _fourier(pts, QB)
    x = (jnp.dot(ff, wpt_ref[...], preferred_element_type=jnp.float32)
         + pooled + ctx_ref[...])
    for l in range(L):
        hdn = jnp.maximum(
            jnp.dot(x, wb1_ref[l], preferred_element_type=jnp.float32), 0.0)
        x = x + jnp.dot(hdn, wb2_ref[l], preferred_element_type=jnp.float32)
    ov_ref[...] = (jnp.dot(x, whv_ref[...],
                           preferred_element_type=jnp.float32) + bhv_ref[...])
    os_ref[...] = (jnp.dot(x, whs_ref[...],
                           preferred_element_type=jnp.float32) + bhs_ref[...])


_trunk = pl.pallas_call(
    _trunk_body,
    grid=(NQ // QB,),
    in_specs=[
        pl.BlockSpec((QB, 3), lambda i: (i, 0)),
        pl.BlockSpec((QB, KMAX), lambda i: (i, 0)),
        pl.BlockSpec((QB, KMAX), lambda i: (i, 0)),
        pl.BlockSpec((QB, KMAX), lambda i: (i, 0)),
        pl.BlockSpec((QB, KMAX), lambda i: (i, 0)),
        pl.BlockSpec((1, H), lambda i: (0, 0)),
        pl.BlockSpec((4, 2 * HL), lambda i: (0, 0)),
        pl.BlockSpec((2 * HL, H), lambda i: (0, 0)),
        pl.BlockSpec((FFP, H), lambda i: (0, 0)),
        pl.BlockSpec((L, H, H), lambda i: (0, 0, 0)),
        pl.BlockSpec((L, H, H), lambda i: (0, 0, 0)),
        pl.BlockSpec((H, 5), lambda i: (0, 0)),
        pl.BlockSpec((1, 5), lambda i: (0, 0)),
        pl.BlockSpec((H, 4), lambda i: (0, 0)),
        pl.BlockSpec((1, 4), lambda i: (0, 0)),
    ],
    out_specs=[
        pl.BlockSpec((QB, 5), lambda i: (i, 0)),
        pl.BlockSpec((QB, 4), lambda i: (i, 0)),
    ],
    out_shape=[
        jax.ShapeDtypeStruct((NQ, 5), jnp.float32),
        jax.ShapeDtypeStruct((NQ, 4), jnp.float32),
    ],
    compiler_params=pltpu.CompilerParams(
        dimension_semantics=("parallel",)),
)


def kernel(geometry_points, surface_points, volume_points, bc_values,
           W_geo, W_bc, W_loc0, W_loc1, W_locproj, W_pt,
           W_blocks1, W_blocks2, W_head_vol, b_head_vol,
           W_head_surf, b_head_surf):
    g = geometry_points[0]
    qs = jnp.concatenate([volume_points[0], surface_points[0]], axis=0)

    nx, ny, nz, nd2 = _sc_ball(
        g[:, 0], g[:, 1], g[:, 2], qs[:, 0], qs[:, 1], qs[:, 2])
    nx = nx.reshape(NQ, KMAX)
    ny = ny.reshape(NQ, KMAX)
    nz = nz.reshape(NQ, KMAX)
    nd2 = nd2.reshape(NQ, KMAX)

    wgeo_pad = jnp.pad(W_geo, ((0, FFP - 51), (0, 0)))
    wpt_pad = jnp.pad(W_pt, ((0, FFP - 51), (0, 0)))
    bc_pad = jnp.pad(bc_values[0], ((0, 4), (0, 0)))
    ctx = _ctx(g, bc_pad, wgeo_pad, W_bc)

    w01 = jnp.concatenate([W_loc0, W_loc1], axis=1)
    ov, osf = _trunk(qs, nx, ny, nz, nd2, ctx,
                     w01, W_locproj, wpt_pad, W_blocks1, W_blocks2,
                     W_head_vol, b_head_vol.reshape(1, 5),
                     W_head_surf, b_head_surf.reshape(1, 4))
    pred_vol = ov[:NV][None]
    pred_surf = osf[NV:][None]
    return (pred_vol, pred_surf)


# speedup vs baseline: 1.4020x; 1.4020x over previous
import functools

import numpy as np
import jax
import jax.numpy as jnp
from jax import lax
from jax.experimental import pallas as pl
from jax.experimental.pallas import tpu as pltpu
from jax.experimental.pallas import tpu_sc as plsc

H = 256
HL = 64
M = 8
L = 4
NG = 4096
NS = 4096
NV = 8192
NQ = NS + NV
R0 = 0.05
R1 = 0.25
R1SQ = R1 * R1
KMAX = 32
BIG = 1e30

NTILES = 32
QPT = NQ // NTILES
CHUNKS = NG // 16
STAGE = NG + 32


GC = 4
NCELL = GC * GC * GC
CAP = 256


def _sc_ball_body(gx_h, gy_h, gz_h, qx_h, qy_h, qz_h,
                  onx_h, ony_h, onz_h, od2_h,
                  gx, gy, gz, qx, qy, qz,
                  obx, oby, obz, obd, sd, si,
                  ids, counts, cellpts):
    cid = lax.axis_index("c")
    sid = lax.axis_index("s")
    wid = sid * 2 + cid
    base = wid * QPT

    pltpu.sync_copy(gx_h, gx)
    pltpu.sync_copy(gy_h, gy)
    pltpu.sync_copy(gz_h, gz)
    pltpu.sync_copy(qx_h.at[pl.ds(base, QPT)], qx.at[pl.ds(0, QPT)])
    pltpu.sync_copy(qy_h.at[pl.ds(base, QPT)], qy.at[pl.ds(0, QPT)])
    pltpu.sync_copy(qz_h.at[pl.ds(base, QPT)], qz.at[pl.ds(0, QPT)])

    iota16 = lax.iota(jnp.int32, 16)
    lane0 = iota16 == 0

    def cell_of(vx, vy, vz):
        cx = (vx * float(GC)).astype(jnp.int32)
        cy = (vy * float(GC)).astype(jnp.int32)
        cz = (vz * float(GC)).astype(jnp.int32)
        return (cx * GC + cy) * GC + cz

    def id_chunk(cc, carry):
        sl = pl.ds(cc * 16, 16)
        ids[sl] = cell_of(gx[sl], gy[sl], gz[sl])
        return carry

    lax.fori_loop(0, CHUNKS, id_chunk, 0)

    def zero_counts(cc, carry):
        counts[pl.ds(cc * 16, 16)] = jnp.zeros((16,), jnp.int32)
        return carry

    lax.fori_loop(0, (NCELL + 16) // 16, zero_counts, 0)

    def place(i, carry):
        c = ids[pl.ds(i, 16)][0]
        cnt = counts[pl.ds(c, 16)][0]
        plsc.store_scatter(cellpts, [jnp.full((16,), c * CAP + cnt)],
                           jnp.full((16,), i), mask=lane0)
        plsc.store_scatter(counts, [jnp.full((16,), c)],
                           jnp.full((16,), cnt + 1), mask=lane0)
        return carry

    lax.fori_loop(0, NG, place, 0)

    def per_query(qi, carry):
        qsl = pl.ds(qi, 16)
        vqx = jnp.full((16,), qx[qsl][0])
        vqy = jnp.full((16,), qy[qsl][0])
        vqz = jnp.full((16,), qz[qsl][0])

        cxv = jnp.clip((vqx * float(GC)).astype(jnp.int32), 0, GC - 1)
        cyv = jnp.clip((vqy * float(GC)).astype(jnp.int32), 0, GC - 1)
        czv = jnp.clip((vqz * float(GC)).astype(jnp.int32), 0, GC - 1)
        cx = cxv[0]
        cy = cyv[0]
        cz = czv[0]
        x0 = jnp.maximum(cx - 1, 0)
        x1 = jnp.minimum(cx + 1, GC - 1)
        y0 = jnp.maximum(cy - 1, 0)
        y1 = jnp.minimum(cy + 1, GC - 1)
        z0 = jnp.maximum(cz - 1, 0)
        z1 = jnp.minimum(cz + 1, GC - 1)

        def scan_cell_chunk(cnt, base_c):
            def chunk_body(ch, off):
                lanes = iota16 + ch * 16
                tm = lanes < cnt
                idxv = jnp.where(tm, cellpts[pl.ds(base_c + ch * 16, 16)], 0)
                sx = plsc.load_gather(gx, [idxv])
                sy = plsc.load_gather(gy, [idxv])
                sz = plsc.load_gather(gz, [idxv])
                dx = sx - vqx
                dy = sy - vqy
                dz = sz - vqz
                d2 = dx * dx + dy * dy + dz * dz
                m = (d2 <= R1SQ) & tm
                cum = plsc.cumsum(m.astype(jnp.int32))
                pos = cum + (off - 1)
                plsc.store_scatter(sd, [pos], d2, mask=m)
                plsc.store_scatter(si, [pos], idxv, mask=m)
                return off + cum[15]
            return chunk_body

        def xbody(xc, offx):
            def ybody(yc, offy):
                def zbody(zc, offz):
                    c = (xc * GC + yc) * GC + zc
                    cnt = counts[pl.ds(c, 16)][0]
                    nch = (cnt + 15) // 16
                    return lax.fori_loop(0, nch,
                                         scan_cell_chunk(cnt, c * CAP), offz)
                return lax.fori_loop(z0, z1 + 1, zbody, offy)
            return lax.fori_loop(y0, y1 + 1, ybody, offx)

        n = lax.fori_loop(x0, x1 + 1, xbody, 0)

        sd[pl.ds(n, 16)] = jnp.full((16,), BIG)
        si[pl.ds(n, 16)] = jnp.zeros((16,), jnp.int32)
        nchunks = (n + 15) // 16

        def merge_chunk(cc, buf):
            a0d, a0i, a1d, a1i = buf
            cd = sd[pl.ds(cc * 16, 16)]
            ci = si[pl.ds(cc * 16, 16)]
            if True:
                cd, ci = plsc.sort_key_val(cd, ci)
                rcd = lax.rev(cd, (0,))
                rci = lax.rev(ci, (0,))
                take = a1d <= rcd
                kd = jnp.where(take, a1d, rcd)
                ki = jnp.where(take, a1i, rci)
                kd, ki = plsc.sort_key_val(kd, ki)
                rkd = lax.rev(kd, (0,))
                rki = lax.rev(ki, (0,))
                t = a0d <= rkd
                ld = jnp.where(t, a0d, rkd)
                li = jnp.where(t, a0i, rki)
                hd = jnp.where(t, rkd, a0d)
                hi = jnp.where(t, rki, a0i)
                a0d, a0i = plsc.sort_key_val(ld, li)
                a1d, a1i = plsc.sort_key_val(hd, hi)
            return (a0d, a0i, a1d, a1i)

        init = (jnp.full((16,), BIG), jnp.zeros((16,), jnp.int32),
                jnp.full((16,), BIG), jnp.zeros((16,), jnp.int32))
        a0d, a0i, a1d, a1i = lax.fori_loop(0, nchunks, merge_chunk, init)

        ob = pl.ds(qi * 32, 16)
        ob2 = pl.ds(qi * 32 + 16, 16)
        obx[ob] = plsc.load_gather(gx, [a0i])
        obx[ob2] = plsc.load_gather(gx, [a1i])
        oby[ob] = plsc.load_gather(gy, [a0i])
        oby[ob2] = plsc.load_gather(gy, [a1i])
        obz[ob] = plsc.load_gather(gz, [a0i])
        obz[ob2] = plsc.load_gather(gz, [a1i])
        obd[ob] = a0d
        obd[ob2] = a1d
        return carry

    lax.fori_loop(0, QPT, per_query, 0)

    out_sl = pl.ds(base * 32, QPT * 32)
    pltpu.sync_copy(obx, onx_h.at[out_sl])
    pltpu.sync_copy(oby, ony_h.at[out_sl])
    pltpu.sync_copy(obz, onz_h.at[out_sl])
    pltpu.sync_copy(obd, od2_h.at[out_sl])


_sc_ball = pl.kernel(
    _sc_ball_body,
    out_type=tuple(jax.ShapeDtypeStruct((NQ * 32,), jnp.float32)
                   for _ in range(4)),
    mesh=plsc.VectorSubcoreMesh(core_axis_name="c", subcore_axis_name="s"),
    compiler_params=pltpu.CompilerParams(needs_layout_passes=False),
    scratch_types=[
        pltpu.VMEM((NG,), jnp.float32),
        pltpu.VMEM((NG,), jnp.float32),
        pltpu.VMEM((NG,), jnp.float32),
        pltpu.VMEM((QPT + 16,), jnp.float32),
        pltpu.VMEM((QPT + 16,), jnp.float32),
        pltpu.VMEM((QPT + 16,), jnp.float32),
        pltpu.VMEM((QPT * 32,), jnp.float32),
        pltpu.VMEM((QPT * 32,), jnp.float32),
        pltpu.VMEM((QPT * 32,), jnp.float32),
        pltpu.VMEM((QPT * 32,), jnp.float32),
        pltpu.VMEM((STAGE,), jnp.float32),
        pltpu.VMEM((STAGE,), jnp.int32),
        pltpu.VMEM((NG + 16,), jnp.int32),
        pltpu.VMEM((NCELL + 16,), jnp.int32),
        pltpu.VMEM((NCELL * CAP,), jnp.int32),
    ],
)


FFP = 64


def _fourier(x, nrows):
    ii = lax.broadcasted_iota(jnp.int32, (1, M), 1)
    freqs = lax.shift_left(1, ii).astype(jnp.float32) * np.float32(np.pi)
    parts = [x]
    for d in range(3):
        ang = x[:, d:d + 1] * freqs
        parts.append(jnp.sin(ang))
        parts.append(jnp.cos(ang))
    parts.append(jnp.zeros((nrows, FFP - 51), jnp.float32))
    return jnp.concatenate(parts, axis=1)


GB = 512


def _ctx_body(geo_ref, bc_ref, wgeo_ref, wbc_ref, out_ref, acc_ref):
    i = pl.program_id(0)

    @pl.when(i == 0)
    def _():
        bc = bc_ref[...]
        hb = jnp.maximum(
            jnp.dot(bc, wbc_ref[...], preferred_element_type=jnp.float32), 0.0)
        acc_ref[...] = jnp.sum(hb, axis=0, keepdims=True) * 0.25

    ff = _fourier(geo_ref[...], GB)
    emb = jnp.dot(ff, wgeo_ref[...], preferred_element_type=jnp.float32)
    acc_ref[...] += jnp.sum(emb, axis=0, keepdims=True) * (1.0 / NG)

    @pl.when(i == pl.num_programs(0) - 1)
    def _():
        out_ref[...] = acc_ref[...]


_ctx = pl.pallas_call(
    _ctx_body,
    grid=(NG // GB,),
    in_specs=[
        pl.BlockSpec((GB, 3), lambda i: (i, 0)),
        pl.BlockSpec((8, 2), lambda i: (0, 0)),
        pl.BlockSpec((FFP, H), lambda i: (0, 0)),
        pl.BlockSpec((2, H), lambda i: (0, 0)),
    ],
    out_specs=pl.BlockSpec((1, H), lambda i: (0, 0)),
    out_shape=jax.ShapeDtypeStruct((1, H), jnp.float32),
    scratch_shapes=[pltpu.VMEM((1, H), jnp.float32)],
    compiler_params=pltpu.CompilerParams(
        dimension_semantics=("arbitrary",)),
)


QB = 256


def _trunk_body(pts_ref, nx_ref, ny_ref, nz_ref, nd2_ref, ctx_ref,
                w01_ref, wproj_ref, wpt_ref, wb1_ref, wb2_ref,
                whv_ref, bhv_ref, whs_ref, bhs_ref,
                ov_ref, os_ref):
    pts = pts_ref[...]
    relx = nx_ref[...] - pts[:, 0:1]
    rely = ny_ref[...] - pts[:, 1:2]
    relz = nz_ref[...] - pts[:, 2:3]
    dist = jnp.sqrt(nd2_ref[...])
    m0 = dist <= R0
    m1 = dist <= R1

    w01 = w01_ref[...]
    acc0 = jnp.zeros((QB, HL), jnp.float32)
    acc1 = jnp.zeros((QB, HL), jnp.float32)
    for s in range(KMAX):
        loc = jnp.concatenate(
            [relx[:, s:s + 1], rely[:, s:s + 1], relz[:, s:s + 1],
             dist[:, s:s + 1]], axis=1)
        enc = jnp.maximum(
            jnp.dot(loc, w01, preferred_element_type=jnp.float32), 0.0)
        mm1 = jnp.broadcast_to(m1[:, s:s + 1], (QB, HL))
        acc1 = jnp.maximum(acc1, jnp.where(mm1, enc[:, HL:], 0.0))
        if s < 8:
            mm0 = jnp.broadcast_to(m0[:, s:s + 1], (QB, HL))
            acc0 = jnp.maximum(acc0, jnp.where(mm0, enc[:, :HL], 0.0))

    pooled = jnp.dot(jnp.concatenate([acc0, acc1], axis=1), wproj_ref[...],
                     preferred_element_type=jnp.float32)
    ff = _fourier(pts, QB)
    x = (jnp.dot(ff, wpt_ref[...], preferred_element_type=jnp.float32)
         + pooled + ctx_ref[...])
    for l in range(L):
        hdn = jnp.maximum(
            jnp.dot(x, wb1_ref[l], preferred_element_type=jnp.float32), 0.0)
        x = x + jnp.dot(hdn, wb2_ref[l], preferred_element_type=jnp.float32)
    ov_ref[...] = (jnp.dot(x, whv_ref[...],
                           preferred_element_type=jnp.float32) + bhv_ref[...])
    os_ref[...] = (jnp.dot(x, whs_ref[...],
                           preferred_element_type=jnp.float32) + bhs_ref[...])


_trunk = pl.pallas_call(
    _trunk_body,
    grid=(NQ // QB,),
    in_specs=[
        pl.BlockSpec((QB, 3), lambda i: (i, 0)),
        pl.BlockSpec((QB, KMAX), lambda i: (i, 0)),
        pl.BlockSpec((QB, KMAX), lambda i: (i, 0)),
        pl.BlockSpec((QB, KMAX), lambda i: (i, 0)),
        pl.BlockSpec((QB, KMAX), lambda i: (i, 0)),
        pl.BlockSpec((1, H), lambda i: (0, 0)),
        pl.BlockSpec((4, 2 * HL), lambda i: (0, 0)),
        pl.BlockSpec((2 * HL, H), lambda i: (0, 0)),
        pl.BlockSpec((FFP, H), lambda i: (0, 0)),
        pl.BlockSpec((L, H, H), lambda i: (0, 0, 0)),
        pl.BlockSpec((L, H, H), lambda i: (0, 0, 0)),
        pl.BlockSpec((H, 5), lambda i: (0, 0)),
        pl.BlockSpec((1, 5), lambda i: (0, 0)),
        pl.BlockSpec((H, 4), lambda i: (0, 0)),
        pl.BlockSpec((1, 4), lambda i: (0, 0)),
    ],
    out_specs=[
        pl.BlockSpec((QB, 5), lambda i: (i, 0)),
        pl.BlockSpec((QB, 4), lambda i: (i, 0)),
    ],
    out_shape=[
        jax.ShapeDtypeStruct((NQ, 5), jnp.float32),
        jax.ShapeDtypeStruct((NQ, 4), jnp.float32),
    ],
    compiler_params=pltpu.CompilerParams(
        dimension_semantics=("parallel",)),
)


def kernel(geometry_points, surface_points, volume_points, bc_values,
           W_geo, W_bc, W_loc0, W_loc1, W_locproj, W_pt,
           W_blocks1, W_blocks2, W_head_vol, b_head_vol,
           W_head_surf, b_head_surf):
    g = geometry_points[0]
    qs = jnp.concatenate([volume_points[0], surface_points[0]], axis=0)

    nx, ny, nz, nd2 = _sc_ball(
        g[:, 0], g[:, 1], g[:, 2], qs[:, 0], qs[:, 1], qs[:, 2])
    nx = nx.reshape(NQ, KMAX)
    ny = ny.reshape(NQ, KMAX)
    nz = nz.reshape(NQ, KMAX)
    nd2 = nd2.reshape(NQ, KMAX)

    wgeo_pad = jnp.pad(W_geo, ((0, FFP - 51), (0, 0)))
    wpt_pad = jnp.pad(W_pt, ((0, FFP - 51), (0, 0)))
    bc_pad = jnp.pad(bc_values[0], ((0, 4), (0, 0)))
    ctx = _ctx(g, bc_pad, wgeo_pad, W_bc)

    w01 = jnp.concatenate([W_loc0, W_loc1], axis=1)
    ov, osf = _trunk(qs, nx, ny, nz, nd2, ctx,
                     w01, W_locproj, wpt_pad, W_blocks1, W_blocks2,
                     W_head_vol, b_head_vol.reshape(1, 5),
                     W_head_surf, b_head_surf.reshape(1, 4))
    pred_vol = ov[:NV][None]
    pred_surf = osf[NV:][None]
    return (pred_vol, pred_surf)


# speedup vs baseline: 1.5751x; 1.1235x over previous
import functools

import numpy as np
import jax
import jax.numpy as jnp
from jax import lax
from jax.experimental import pallas as pl
from jax.experimental.pallas import tpu as pltpu
from jax.experimental.pallas import tpu_sc as plsc

H = 256
HL = 64
M = 8
L = 4
NG = 4096
NS = 4096
NV = 8192
NQ = NS + NV
R0 = 0.05
R1 = 0.25
R1SQ = R1 * R1
KMAX = 32
BIG = 1e30

NTILES = 32
QPT = NQ // NTILES
CHUNKS = NG // 16
STAGE = NG + 32


GC = 4
NCELL = GC * GC * GC
CAP = 256


def _sc_ball_body(gx_h, gy_h, gz_h, qx_h, qy_h, qz_h,
                  onx_h, ony_h, onz_h, od2_h,
                  gx, gy, gz, qx, qy, qz,
                  obx, oby, obz, obd, sd, si,
                  ids, counts, cellpts):
    cid = lax.axis_index("c")
    sid = lax.axis_index("s")
    wid = sid * 2 + cid
    base = wid * QPT

    pltpu.sync_copy(gx_h, gx)
    pltpu.sync_copy(gy_h, gy)
    pltpu.sync_copy(gz_h, gz)
    pltpu.sync_copy(qx_h.at[pl.ds(base, QPT)], qx.at[pl.ds(0, QPT)])
    pltpu.sync_copy(qy_h.at[pl.ds(base, QPT)], qy.at[pl.ds(0, QPT)])
    pltpu.sync_copy(qz_h.at[pl.ds(base, QPT)], qz.at[pl.ds(0, QPT)])

    iota16 = lax.iota(jnp.int32, 16)
    lane0 = iota16 == 0

    def cell_of(vx, vy, vz):
        cx = (vx * float(GC)).astype(jnp.int32)
        cy = (vy * float(GC)).astype(jnp.int32)
        cz = (vz * float(GC)).astype(jnp.int32)
        return (cx * GC + cy) * GC + cz

    def id_chunk(cc, carry):
        sl = pl.ds(cc * 16, 16)
        ids[sl] = cell_of(gx[sl], gy[sl], gz[sl])
        return carry

    lax.fori_loop(0, CHUNKS, id_chunk, 0)

    def zero_counts(cc, carry):
        counts[pl.ds(cc * 16, 16)] = jnp.zeros((16,), jnp.int32)
        return carry

    lax.fori_loop(0, (NCELL + 16) // 16, zero_counts, 0)

    def place(i, carry):
        c = ids[pl.ds(i, 16)][0]
        cnt = counts[pl.ds(c, 16)][0]
        plsc.store_scatter(cellpts, [jnp.full((16,), c * CAP + cnt)],
                           jnp.full((16,), i), mask=lane0)
        plsc.store_scatter(counts, [jnp.full((16,), c)],
                           jnp.full((16,), cnt + 1), mask=lane0)
        return carry

    lax.fori_loop(0, NG, place, 0)

    def per_query(qi, carry):
        qsl = pl.ds(qi, 16)
        vqx = jnp.full((16,), qx[qsl][0])
        vqy = jnp.full((16,), qy[qsl][0])
        vqz = jnp.full((16,), qz[qsl][0])

        cxv = jnp.clip((vqx * float(GC)).astype(jnp.int32), 0, GC - 1)
        cyv = jnp.clip((vqy * float(GC)).astype(jnp.int32), 0, GC - 1)
        czv = jnp.clip((vqz * float(GC)).astype(jnp.int32), 0, GC - 1)
        cx = cxv[0]
        cy = cyv[0]
        cz = czv[0]
        x0 = jnp.maximum(cx - 1, 0)
        x1 = jnp.minimum(cx + 1, GC - 1)
        y0 = jnp.maximum(cy - 1, 0)
        y1 = jnp.minimum(cy + 1, GC - 1)
        z0 = jnp.maximum(cz - 1, 0)
        z1 = jnp.minimum(cz + 1, GC - 1)

        def scan_cell_chunk(cnt, base_c):
            def chunk_body(ch, off):
                lanes = iota16 + ch * 16
                tm = lanes < cnt
                idxv = jnp.where(tm, cellpts[pl.ds(base_c + ch * 16, 16)], 0)
                sx = plsc.load_gather(gx, [idxv])
                sy = plsc.load_gather(gy, [idxv])
                sz = plsc.load_gather(gz, [idxv])
                dx = sx - vqx
                dy = sy - vqy
                dz = sz - vqz
                d2 = dx * dx + dy * dy + dz * dz
                m = (d2 <= R1SQ) & tm
                cum = plsc.cumsum(m.astype(jnp.int32))
                pos = cum + (off - 1)
                plsc.store_scatter(sd, [pos], d2, mask=m)
                plsc.store_scatter(si, [pos], idxv, mask=m)
                return off + cum[15]
            return chunk_body

        def xbody(xc, offx):
            def ybody(yc, offy):
                def zbody(zc, offz):
                    c = (xc * GC + yc) * GC + zc
                    cnt = counts[pl.ds(c, 16)][0]
                    nch = (cnt + 15) // 16
                    return lax.fori_loop(0, nch,
                                         scan_cell_chunk(cnt, c * CAP), offz)
                return lax.fori_loop(z0, z1 + 1, zbody, offy)
            return lax.fori_loop(y0, y1 + 1, ybody, offx)

        n = lax.fori_loop(x0, x1 + 1, xbody, 0)

        sd[pl.ds(n, 16)] = jnp.full((16,), BIG)
        si[pl.ds(n, 16)] = jnp.zeros((16,), jnp.int32)
        nchunks = (n + 15) // 16

        def merge_chunk(cc, buf):
            a0d, a0i, a1d, a1i = buf
            cd = sd[pl.ds(cc * 16, 16)]
            ci = si[pl.ds(cc * 16, 16)]
            if True:
                cd, ci = plsc.sort_key_val(cd, ci)
                rcd = lax.rev(cd, (0,))
                rci = lax.rev(ci, (0,))
                take = a1d <= rcd
                kd = jnp.where(take, a1d, rcd)
                ki = jnp.where(take, a1i, rci)
                kd, ki = plsc.sort_key_val(kd, ki)
                rkd = lax.rev(kd, (0,))
                rki = lax.rev(ki, (0,))
                t = a0d <= rkd
                ld = jnp.where(t, a0d, rkd)
                li = jnp.where(t, a0i, rki)
                hd = jnp.where(t, rkd, a0d)
                hi = jnp.where(t, rki, a0i)
                a0d, a0i = plsc.sort_key_val(ld, li)
                a1d, a1i = plsc.sort_key_val(hd, hi)
            return (a0d, a0i, a1d, a1i)

        init = (jnp.full((16,), BIG), jnp.zeros((16,), jnp.int32),
                jnp.full((16,), BIG), jnp.zeros((16,), jnp.int32))
        a0d, a0i, a1d, a1i = lax.fori_loop(0, nchunks, merge_chunk, init)

        ob = pl.ds(qi * 32, 16)
        ob2 = pl.ds(qi * 32 + 16, 16)
        obx[ob] = plsc.load_gather(gx, [a0i])
        obx[ob2] = plsc.load_gather(gx, [a1i])
        oby[ob] = plsc.load_gather(gy, [a0i])
        oby[ob2] = plsc.load_gather(gy, [a1i])
        obz[ob] = plsc.load_gather(gz, [a0i])
        obz[ob2] = plsc.load_gather(gz, [a1i])
        obd[ob] = a0d
        obd[ob2] = a1d
        return carry

    lax.fori_loop(0, QPT, per_query, 0)

    out_sl = pl.ds(base * 32, QPT * 32)
    pltpu.sync_copy(obx, onx_h.at[out_sl])
    pltpu.sync_copy(oby, ony_h.at[out_sl])
    pltpu.sync_copy(obz, onz_h.at[out_sl])
    pltpu.sync_copy(obd, od2_h.at[out_sl])


_sc_ball = pl.kernel(
    _sc_ball_body,
    out_type=tuple(jax.ShapeDtypeStruct((NQ * 32,), jnp.float32)
                   for _ in range(4)),
    mesh=plsc.VectorSubcoreMesh(core_axis_name="c", subcore_axis_name="s"),
    compiler_params=pltpu.CompilerParams(needs_layout_passes=False),
    scratch_types=[
        pltpu.VMEM((NG,), jnp.float32),
        pltpu.VMEM((NG,), jnp.float32),
        pltpu.VMEM((NG,), jnp.float32),
        pltpu.VMEM((QPT + 16,), jnp.float32),
        pltpu.VMEM((QPT + 16,), jnp.float32),
        pltpu.VMEM((QPT + 16,), jnp.float32),
        pltpu.VMEM((QPT * 32,), jnp.float32),
        pltpu.VMEM((QPT * 32,), jnp.float32),
        pltpu.VMEM((QPT * 32,), jnp.float32),
        pltpu.VMEM((QPT * 32,), jnp.float32),
        pltpu.VMEM((STAGE,), jnp.float32),
        pltpu.VMEM((STAGE,), jnp.int32),
        pltpu.VMEM((NG + 16,), jnp.int32),
        pltpu.VMEM((NCELL + 16,), jnp.int32),
        pltpu.VMEM((NCELL * CAP,), jnp.int32),
    ],
)


FFP = 64


def _fourier(x, nrows):
    ii = lax.broadcasted_iota(jnp.int32, (1, M), 1)
    freqs = lax.shift_left(1, ii).astype(jnp.float32) * np.float32(np.pi)
    parts = [x]
    for d in range(3):
        ang = x[:, d:d + 1] * freqs
        parts.append(jnp.sin(ang))
        parts.append(jnp.cos(ang))
    parts.append(jnp.zeros((nrows, FFP - 51), jnp.float32))
    return jnp.concatenate(parts, axis=1)


GB = 512


def _ctx_body(geo_ref, bc_ref, wgeo_ref, wbc_ref, out_ref, acc_ref):
    i = pl.program_id(0)

    @pl.when(i == 0)
    def _():
        bc = bc_ref[...]
        hb = jnp.maximum(
            jnp.dot(bc, wbc_ref[...], preferred_element_type=jnp.float32), 0.0)
        acc_ref[...] = jnp.sum(hb, axis=0, keepdims=True) * 0.25

    ff = _fourier(geo_ref[...], GB)
    emb = jnp.dot(ff, wgeo_ref[...], preferred_element_type=jnp.float32)
    acc_ref[...] += jnp.sum(emb, axis=0, keepdims=True) * (1.0 / NG)

    @pl.when(i == pl.num_programs(0) - 1)
    def _():
        out_ref[...] = acc_ref[...]


_ctx = pl.pallas_call(
    _ctx_body,
    grid=(NG // GB,),
    in_specs=[
        pl.BlockSpec((GB, 3), lambda i: (i, 0)),
        pl.BlockSpec((8, 2), lambda i: (0, 0)),
        pl.BlockSpec((FFP, H), lambda i: (0, 0)),
        pl.BlockSpec((2, H), lambda i: (0, 0)),
    ],
    out_specs=pl.BlockSpec((1, H), lambda i: (0, 0)),
    out_shape=jax.ShapeDtypeStruct((1, H), jnp.float32),
    scratch_shapes=[pltpu.VMEM((1, H), jnp.float32)],
    compiler_params=pltpu.CompilerParams(
        dimension_semantics=("arbitrary",)),
)


QB = 256


def _trunk_body(pts_ref, nx_ref, ny_ref, nz_ref, nd2_ref, ctx_ref,
                w01_ref, wproj_ref, wpt_ref, wb1_ref, wb2_ref,
                whv_ref, bhv_ref, whs_ref, bhs_ref,
                ov_ref, os_ref):
    pts = pts_ref[...]
    relx = nx_ref[...] - pts[:, 0:1]
    rely = ny_ref[...] - pts[:, 1:2]
    relz = nz_ref[...] - pts[:, 2:3]
    dist = jnp.sqrt(nd2_ref[...])
    m0 = dist <= R0
    m1 = dist <= R1

    loc128 = jnp.concatenate([relx, rely, relz, dist], axis=1)
    enc_all = jnp.maximum(
        jnp.dot(loc128, w01_ref[...], preferred_element_type=jnp.float32),
        0.0)
    acc0 = jnp.zeros((QB, HL), jnp.float32)
    acc1 = jnp.zeros((QB, HL), jnp.float32)
    for s in range(KMAX):
        enc = enc_all[:, s * 2 * HL:(s + 1) * 2 * HL]
        mm1 = jnp.broadcast_to(m1[:, s:s + 1], (QB, HL))
        acc1 = jnp.maximum(acc1, jnp.where(mm1, enc[:, HL:], 0.0))
        if s < 8:
            mm0 = jnp.broadcast_to(m0[:, s:s + 1], (QB, HL))
            acc0 = jnp.maximum(acc0, jnp.where(mm0, enc[:, :HL], 0.0))

    pooled = jnp.dot(jnp.concatenate([acc0, acc1], axis=1), wproj_ref[...],
                     preferred_element_type=jnp.float32)
    ff = _fourier(pts, QB)
    x = (jnp.dot(ff, wpt_ref[...], preferred_element_type=jnp.float32)
         + pooled + ctx_ref[...])
    for l in range(L):
        hdn = jnp.maximum(
            jnp.dot(x, wb1_ref[l], preferred_element_type=jnp.float32), 0.0)
        x = x + jnp.dot(hdn, wb2_ref[l], preferred_element_type=jnp.float32)
    ov_ref[...] = (jnp.dot(x, whv_ref[...],
                           preferred_element_type=jnp.float32) + bhv_ref[...])
    os_ref[...] = (jnp.dot(x, whs_ref[...],
                           preferred_element_type=jnp.float32) + bhs_ref[...])


_trunk = pl.pallas_call(
    _trunk_body,
    grid=(NQ // QB,),
    in_specs=[
        pl.BlockSpec((QB, 3), lambda i: (i, 0)),
        pl.BlockSpec((QB, KMAX), lambda i: (i, 0)),
        pl.BlockSpec((QB, KMAX), lambda i: (i, 0)),
        pl.BlockSpec((QB, KMAX), lambda i: (i, 0)),
        pl.BlockSpec((QB, KMAX), lambda i: (i, 0)),
        pl.BlockSpec((1, H), lambda i: (0, 0)),
        pl.BlockSpec((2 * HL, KMAX * 2 * HL), lambda i: (0, 0)),
        pl.BlockSpec((2 * HL, H), lambda i: (0, 0)),
        pl.BlockSpec((FFP, H), lambda i: (0, 0)),
        pl.BlockSpec((L, H, H), lambda i: (0, 0, 0)),
        pl.BlockSpec((L, H, H), lambda i: (0, 0, 0)),
        pl.BlockSpec((H, 5), lambda i: (0, 0)),
        pl.BlockSpec((1, 5), lambda i: (0, 0)),
        pl.BlockSpec((H, 4), lambda i: (0, 0)),
        pl.BlockSpec((1, 4), lambda i: (0, 0)),
    ],
    out_specs=[
        pl.BlockSpec((QB, 5), lambda i: (i, 0)),
        pl.BlockSpec((QB, 4), lambda i: (i, 0)),
    ],
    out_shape=[
        jax.ShapeDtypeStruct((NQ, 5), jnp.float32),
        jax.ShapeDtypeStruct((NQ, 4), jnp.float32),
    ],
    compiler_params=pltpu.CompilerParams(
        dimension_semantics=("parallel",)),
)


def kernel(geometry_points, surface_points, volume_points, bc_values,
           W_geo, W_bc, W_loc0, W_loc1, W_locproj, W_pt,
           W_blocks1, W_blocks2, W_head_vol, b_head_vol,
           W_head_surf, b_head_surf):
    g = geometry_points[0]
    qs = jnp.concatenate([volume_points[0], surface_points[0]], axis=0)

    nx, ny, nz, nd2 = _sc_ball(
        g[:, 0], g[:, 1], g[:, 2], qs[:, 0], qs[:, 1], qs[:, 2])
    nx = nx.reshape(NQ, KMAX)
    ny = ny.reshape(NQ, KMAX)
    nz = nz.reshape(NQ, KMAX)
    nd2 = nd2.reshape(NQ, KMAX)

    wgeo_pad = jnp.pad(W_geo, ((0, FFP - 51), (0, 0)))
    wpt_pad = jnp.pad(W_pt, ((0, FFP - 51), (0, 0)))
    bc_pad = jnp.pad(bc_values[0], ((0, 4), (0, 0)))
    ctx = _ctx(g, bc_pad, wgeo_pad, W_bc)

    w01 = jnp.concatenate([W_loc0, W_loc1], axis=1)
    eye32 = jnp.eye(KMAX, dtype=jnp.float32)
    w_bd = (w01[:, None, None, :] * eye32[None, :, :, None]).reshape(
        2 * HL, KMAX * 2 * HL)
    ov, osf = _trunk(qs, nx, ny, nz, nd2, ctx,
                     w_bd, W_locproj, wpt_pad, W_blocks1, W_blocks2,
                     W_head_vol, b_head_vol.reshape(1, 5),
                     W_head_surf, b_head_surf.reshape(1, 4))
    pred_vol = ov[:NV][None]
    pred_surf = osf[NV:][None]
    return (pred_vol, pred_surf)


# speedup vs baseline: 1.5754x; 1.0002x over previous
import functools

import numpy as np
import jax
import jax.numpy as jnp
from jax import lax
from jax.experimental import pallas as pl
from jax.experimental.pallas import tpu as pltpu
from jax.experimental.pallas import tpu_sc as plsc

H = 256
HL = 64
M = 8
L = 4
NG = 4096
NS = 4096
NV = 8192
NQ = NS + NV
R0 = 0.05
R1 = 0.25
R1SQ = R1 * R1
KMAX = 32
BIG = 1e30

NTILES = 32
QPT = NQ // NTILES
CHUNKS = NG // 16
STAGE = NG + 32


GC = 4
NCELL = GC * GC * GC
CAP = 256


def _sc_ball_body(gx_h, gy_h, gz_h, qx_h, qy_h, qz_h,
                  onx_h, ony_h, onz_h, od2_h,
                  gx, gy, gz, qx, qy, qz,
                  obx, oby, obz, obd, sd, si,
                  ids, counts, cellpts):
    cid = lax.axis_index("c")
    sid = lax.axis_index("s")
    wid = sid * 2 + cid
    base = wid * QPT

    pltpu.sync_copy(gx_h, gx)
    pltpu.sync_copy(gy_h, gy)
    pltpu.sync_copy(gz_h, gz)
    pltpu.sync_copy(qx_h.at[pl.ds(base, QPT)], qx.at[pl.ds(0, QPT)])
    pltpu.sync_copy(qy_h.at[pl.ds(base, QPT)], qy.at[pl.ds(0, QPT)])
    pltpu.sync_copy(qz_h.at[pl.ds(base, QPT)], qz.at[pl.ds(0, QPT)])

    iota16 = lax.iota(jnp.int32, 16)
    lane0 = iota16 == 0

    def cell_of(vx, vy, vz):
        cx = (vx * float(GC)).astype(jnp.int32)
        cy = (vy * float(GC)).astype(jnp.int32)
        cz = (vz * float(GC)).astype(jnp.int32)
        return (cx * GC + cy) * GC + cz

    def id_chunk(cc, carry):
        sl = pl.ds(cc * 16, 16)
        ids[sl] = cell_of(gx[sl], gy[sl], gz[sl])
        return carry

    lax.fori_loop(0, CHUNKS, id_chunk, 0)

    def zero_counts(cc, carry):
        counts[pl.ds(cc * 16, 16)] = jnp.zeros((16,), jnp.int32)
        return carry

    lax.fori_loop(0, (NCELL + 16) // 16, zero_counts, 0)

    def place(i, carry):
        c = ids[pl.ds(i, 16)][0]
        cnt = counts[pl.ds(c, 16)][0]
        plsc.store_scatter(cellpts, [jnp.full((16,), c * CAP + cnt)],
                           jnp.full((16,), i), mask=lane0)
        plsc.store_scatter(counts, [jnp.full((16,), c)],
                           jnp.full((16,), cnt + 1), mask=lane0)
        return carry

    lax.fori_loop(0, NG, place, 0)

    def per_query(qi, carry):
        qsl = pl.ds(qi, 16)
        vqx = jnp.full((16,), qx[qsl][0])
        vqy = jnp.full((16,), qy[qsl][0])
        vqz = jnp.full((16,), qz[qsl][0])

        cxv = jnp.clip((vqx * float(GC)).astype(jnp.int32), 0, GC - 1)
        cyv = jnp.clip((vqy * float(GC)).astype(jnp.int32), 0, GC - 1)
        czv = jnp.clip((vqz * float(GC)).astype(jnp.int32), 0, GC - 1)
        cx = cxv[0]
        cy = cyv[0]
        cz = czv[0]
        x0 = jnp.maximum(cx - 1, 0)
        x1 = jnp.minimum(cx + 1, GC - 1)
        y0 = jnp.maximum(cy - 1, 0)
        y1 = jnp.minimum(cy + 1, GC - 1)
        z0 = jnp.maximum(cz - 1, 0)
        z1 = jnp.minimum(cz + 1, GC - 1)

        def scan_cell_chunk(cnt, base_c):
            def chunk_body(ch, off):
                lanes = iota16 + ch * 16
                tm = lanes < cnt
                idxv = jnp.where(tm, cellpts[pl.ds(base_c + ch * 16, 16)], 0)
                sx = plsc.load_gather(gx, [idxv])
                sy = plsc.load_gather(gy, [idxv])
                sz = plsc.load_gather(gz, [idxv])
                dx = sx - vqx
                dy = sy - vqy
                dz = sz - vqz
                d2 = dx * dx + dy * dy + dz * dz
                m = (d2 <= R1SQ) & tm
                cum = plsc.cumsum(m.astype(jnp.int32))
                pos = cum + (off - 1)
                plsc.store_scatter(sd, [pos], d2, mask=m)
                plsc.store_scatter(si, [pos], idxv, mask=m)
                return off + cum[15]
            return chunk_body

        def xbody(xc, offx):
            def ybody(yc, offy):
                def zbody(zc, offz):
                    c = (xc * GC + yc) * GC + zc
                    cnt = counts[pl.ds(c, 16)][0]
                    nch = (cnt + 15) // 16
                    return lax.fori_loop(0, nch,
                                         scan_cell_chunk(cnt, c * CAP), offz)
                return lax.fori_loop(z0, z1 + 1, zbody, offy)
            return lax.fori_loop(y0, y1 + 1, ybody, offx)

        n = lax.fori_loop(x0, x1 + 1, xbody, 0)

        sd[pl.ds(n, 16)] = jnp.full((16,), BIG)
        si[pl.ds(n, 16)] = jnp.zeros((16,), jnp.int32)
        nchunks = (n + 15) // 16

        def merge_chunk(cc, buf):
            a0d, a0i, a1d, a1i = buf
            cd = sd[pl.ds(cc * 16, 16)]
            ci = si[pl.ds(cc * 16, 16)]
            cd, ci = plsc.sort_key_val(cd, ci)
            rcd = lax.rev(cd, (0,))
            rci = lax.rev(ci, (0,))
            take = a1d <= rcd
            kd = jnp.where(take, a1d, rcd)
            ki = jnp.where(take, a1i, rci)
            kd, ki = plsc.sort_key_val(kd, ki)
            rkd = lax.rev(kd, (0,))
            rki = lax.rev(ki, (0,))
            t = a0d <= rkd
            ld = jnp.where(t, a0d, rkd)
            li = jnp.where(t, a0i, rki)
            hd = jnp.where(t, rkd, a0d)
            hi = jnp.where(t, rki, a0i)
            a0d, a0i = plsc.sort_key_val(ld, li)
            a1d, a1i = plsc.sort_key_val(hd, hi)
            return (a0d, a0i, a1d, a1i)

        init = (jnp.full((16,), BIG), jnp.zeros((16,), jnp.int32),
                jnp.full((16,), BIG), jnp.zeros((16,), jnp.int32))
        a0d, a0i, a1d, a1i = lax.fori_loop(0, nchunks, merge_chunk, init)

        ob = pl.ds(qi * 32, 16)
        ob2 = pl.ds(qi * 32 + 16, 16)
        obx[ob] = plsc.load_gather(gx, [a0i])
        obx[ob2] = plsc.load_gather(gx, [a1i])
        oby[ob] = plsc.load_gather(gy, [a0i])
        oby[ob2] = plsc.load_gather(gy, [a1i])
        obz[ob] = plsc.load_gather(gz, [a0i])
        obz[ob2] = plsc.load_gather(gz, [a1i])
        obd[ob] = a0d
        obd[ob2] = a1d
        return carry

    lax.fori_loop(0, QPT, per_query, 0)

    out_sl = pl.ds(base * 32, QPT * 32)
    pltpu.sync_copy(obx, onx_h.at[out_sl])
    pltpu.sync_copy(oby, ony_h.at[out_sl])
    pltpu.sync_copy(obz, onz_h.at[out_sl])
    pltpu.sync_copy(obd, od2_h.at[out_sl])


_sc_ball = pl.kernel(
    _sc_ball_body,
    out_type=tuple(jax.ShapeDtypeStruct((NQ * 32,), jnp.float32)
                   for _ in range(4)),
    mesh=plsc.VectorSubcoreMesh(core_axis_name="c", subcore_axis_name="s"),
    compiler_params=pltpu.CompilerParams(needs_layout_passes=False),
    scratch_types=[
        pltpu.VMEM((NG,), jnp.float32),
        pltpu.VMEM((NG,), jnp.float32),
        pltpu.VMEM((NG,), jnp.float32),
        pltpu.VMEM((QPT + 16,), jnp.float32),
        pltpu.VMEM((QPT + 16,), jnp.float32),
        pltpu.VMEM((QPT + 16,), jnp.float32),
        pltpu.VMEM((QPT * 32,), jnp.float32),
        pltpu.VMEM((QPT * 32,), jnp.float32),
        pltpu.VMEM((QPT * 32,), jnp.float32),
        pltpu.VMEM((QPT * 32,), jnp.float32),
        pltpu.VMEM((STAGE,), jnp.float32),
        pltpu.VMEM((STAGE,), jnp.int32),
        pltpu.VMEM((NG + 16,), jnp.int32),
        pltpu.VMEM((NCELL + 16,), jnp.int32),
        pltpu.VMEM((NCELL * CAP,), jnp.int32),
    ],
)


FFP = 64


def _fourier(x, nrows):
    ii = lax.broadcasted_iota(jnp.int32, (1, M), 1)
    freqs = lax.shift_left(1, ii).astype(jnp.float32) * np.float32(np.pi)
    parts = [x]
    for d in range(3):
        ang = x[:, d:d + 1] * freqs
        parts.append(jnp.sin(ang))
        parts.append(jnp.cos(ang))
    parts.append(jnp.zeros((nrows, FFP - 51), jnp.float32))
    return jnp.concatenate(parts, axis=1)


GB = 512


def _ctx_body(geo_ref, bc_ref, wgeo_ref, wbc_ref, out_ref, acc_ref):
    i = pl.program_id(0)

    @pl.when(i == 0)
    def _():
        bc = bc_ref[...]
        hb = jnp.maximum(
            jnp.dot(bc, wbc_ref[...], preferred_element_type=jnp.float32), 0.0)
        acc_ref[...] = jnp.sum(hb, axis=0, keepdims=True) * 0.25

    ff = _fourier(geo_ref[...], GB)
    emb = jnp.dot(ff, wgeo_ref[...], preferred_element_type=jnp.float32)
    acc_ref[...] += jnp.sum(emb, axis=0, keepdims=True) * (1.0 / NG)

    @pl.when(i == pl.num_programs(0) - 1)
    def _():
        out_ref[...] = acc_ref[...]


_ctx = pl.pallas_call(
    _ctx_body,
    grid=(NG // GB,),
    in_specs=[
        pl.BlockSpec((GB, 3), lambda i: (i, 0)),
        pl.BlockSpec((8, 2), lambda i: (0, 0)),
        pl.BlockSpec((FFP, H), lambda i: (0, 0)),
        pl.BlockSpec((2, H), lambda i: (0, 0)),
    ],
    out_specs=pl.BlockSpec((1, H), lambda i: (0, 0)),
    out_shape=jax.ShapeDtypeStruct((1, H), jnp.float32),
    scratch_shapes=[pltpu.VMEM((1, H), jnp.float32)],
    compiler_params=pltpu.CompilerParams(
        dimension_semantics=("arbitrary",)),
)


QB = 256


def _trunk_body(pts_ref, nx_ref, ny_ref, nz_ref, nd2_ref, ctx_ref,
                w01_ref, wproj_ref, wpt_ref, wb1_ref, wb2_ref,
                whv_ref, bhv_ref, whs_ref, bhs_ref,
                ov_ref, os_ref):
    pts = pts_ref[...]
    relx = nx_ref[...] - pts[:, 0:1]
    rely = ny_ref[...] - pts[:, 1:2]
    relz = nz_ref[...] - pts[:, 2:3]
    dist = jnp.sqrt(nd2_ref[...])
    m0 = dist <= R0
    m1 = dist <= R1

    loc128 = jnp.concatenate([relx, rely, relz, dist], axis=1)
    enc_all = jnp.maximum(
        jnp.dot(loc128, w01_ref[...], preferred_element_type=jnp.float32),
        0.0)
    acc0 = jnp.zeros((QB, HL), jnp.float32)
    acc1 = jnp.zeros((QB, HL), jnp.float32)
    for s in range(KMAX):
        enc = enc_all[:, s * 2 * HL:(s + 1) * 2 * HL]
        mm1 = jnp.broadcast_to(m1[:, s:s + 1], (QB, HL))
        acc1 = jnp.maximum(acc1, jnp.where(mm1, enc[:, HL:], 0.0))
        if s < 8:
            mm0 = jnp.broadcast_to(m0[:, s:s + 1], (QB, HL))
            acc0 = jnp.maximum(acc0, jnp.where(mm0, enc[:, :HL], 0.0))

    pooled = jnp.dot(jnp.concatenate([acc0, acc1], axis=1), wproj_ref[...],
                     preferred_element_type=jnp.float32)
    ff = _fourier(pts, QB)
    x = (jnp.dot(ff, wpt_ref[...], preferred_element_type=jnp.float32)
         + pooled + ctx_ref[...])
    for l in range(L):
        hdn = jnp.maximum(
            jnp.dot(x, wb1_ref[l], preferred_element_type=jnp.float32), 0.0)
        x = x + jnp.dot(hdn, wb2_ref[l], preferred_element_type=jnp.float32)
    ov_ref[...] = (jnp.dot(x, whv_ref[...],
                           preferred_element_type=jnp.float32) + bhv_ref[...])
    os_ref[...] = (jnp.dot(x, whs_ref[...],
                           preferred_element_type=jnp.float32) + bhs_ref[...])


_trunk = pl.pallas_call(
    _trunk_body,
    grid=(NQ // QB,),
    in_specs=[
        pl.BlockSpec((QB, 3), lambda i: (i, 0)),
        pl.BlockSpec((QB, KMAX), lambda i: (i, 0)),
        pl.BlockSpec((QB, KMAX), lambda i: (i, 0)),
        pl.BlockSpec((QB, KMAX), lambda i: (i, 0)),
        pl.BlockSpec((QB, KMAX), lambda i: (i, 0)),
        pl.BlockSpec((1, H), lambda i: (0, 0)),
        pl.BlockSpec((2 * HL, KMAX * 2 * HL), lambda i: (0, 0)),
        pl.BlockSpec((2 * HL, H), lambda i: (0, 0)),
        pl.BlockSpec((FFP, H), lambda i: (0, 0)),
        pl.BlockSpec((L, H, H), lambda i: (0, 0, 0)),
        pl.BlockSpec((L, H, H), lambda i: (0, 0, 0)),
        pl.BlockSpec((H, 5), lambda i: (0, 0)),
        pl.BlockSpec((1, 5), lambda i: (0, 0)),
        pl.BlockSpec((H, 4), lambda i: (0, 0)),
        pl.BlockSpec((1, 4), lambda i: (0, 0)),
    ],
    out_specs=[
        pl.BlockSpec((QB, 5), lambda i: (i, 0)),
        pl.BlockSpec((QB, 4), lambda i: (i, 0)),
    ],
    out_shape=[
        jax.ShapeDtypeStruct((NQ, 5), jnp.float32),
        jax.ShapeDtypeStruct((NQ, 4), jnp.float32),
    ],
    compiler_params=pltpu.CompilerParams(
        dimension_semantics=("parallel",)),
)


def kernel(geometry_points, surface_points, volume_points, bc_values,
           W_geo, W_bc, W_loc0, W_loc1, W_locproj, W_pt,
           W_blocks1, W_blocks2, W_head_vol, b_head_vol,
           W_head_surf, b_head_surf):
    g = geometry_points[0]
    qs = jnp.concatenate([volume_points[0], surface_points[0]], axis=0)

    nx, ny, nz, nd2 = _sc_ball(
        g[:, 0], g[:, 1], g[:, 2], qs[:, 0], qs[:, 1], qs[:, 2])
    nx = nx.reshape(NQ, KMAX)
    ny = ny.reshape(NQ, KMAX)
    nz = nz.reshape(NQ, KMAX)
    nd2 = nd2.reshape(NQ, KMAX)

    wgeo_pad = jnp.pad(W_geo, ((0, FFP - 51), (0, 0)))
    wpt_pad = jnp.pad(W_pt, ((0, FFP - 51), (0, 0)))
    bc_pad = jnp.pad(bc_values[0], ((0, 4), (0, 0)))
    ctx = _ctx(g, bc_pad, wgeo_pad, W_bc)

    w01 = jnp.concatenate([W_loc0, W_loc1], axis=1)
    eye32 = jnp.eye(KMAX, dtype=jnp.float32)
    w_bd = (w01[:, None, None, :] * eye32[None, :, :, None]).reshape(
        2 * HL, KMAX * 2 * HL)
    ov, osf = _trunk(qs, nx, ny, nz, nd2, ctx,
                     w_bd, W_locproj, wpt_pad, W_blocks1, W_blocks2,
                     W_head_vol, b_head_vol.reshape(1, 5),
                     W_head_surf, b_head_surf.reshape(1, 4))
    pred_vol = ov[:NV][None]
    pred_surf = osf[NV:][None]
    return (pred_vol, pred_surf)
